# baseline (speedup 1.0000x reference)
import functools

import jax
import jax.numpy as jnp
from jax import lax
from jax.experimental import pallas as pl
from jax.experimental.pallas import tpu as pltpu
from jax.experimental.pallas import tpu_sc as plsc

_NUM_CORES = 2
_NUM_SUBCORES = 16
_NW = _NUM_CORES * _NUM_SUBCORES
_CHUNK = 128


@functools.partial(jax.jit, static_argnums=(3, 4, 5))
def _gather_rows(idx, cos_i, sin_i, n_ch, chunk, dw):
    n = _NW * n_ch * chunk
    per_w = n_ch * chunk
    n_pass = 2
    ch_pp = n_ch // n_pass
    rows_pp = ch_pp * chunk
    out_t = jax.ShapeDtypeStruct((n, dw), jnp.int32)
    mesh = plsc.VectorSubcoreMesh(core_axis_name="c", subcore_axis_name="s")

    @functools.partial(
        pl.kernel,
        mesh=mesh,
        out_type=(out_t, out_t),
        scratch_types=[
            pltpu.VMEM((n_ch, chunk), jnp.int32),
            pltpu.VMEM((rows_pp, dw), jnp.int32),
            pltpu.VMEM((rows_pp, dw), jnp.int32),
            pltpu.SemaphoreType.DMA,
        ],
        compiler_params=pltpu.CompilerParams(use_tc_tiling_on_sc=False),
    )
    def body(idx_hbm, cos_hbm, sin_hbm, cos_out, sin_out, idx_v, cos_v, sin_v, sem):
        wid = lax.axis_index("s") * _NUM_CORES + lax.axis_index("c")
        pltpu.sync_copy(idx_hbm.at[wid], idx_v)
        base = wid * per_w
        for p in range(n_pass):
            copies = []
            for j in range(ch_pp):
                c = p * ch_pp + j
                copies.append(pltpu.async_copy(
                    cos_hbm.at[idx_v.at[c]], cos_v.at[pl.ds(j * chunk, chunk)], sem))
                copies.append(pltpu.async_copy(
                    sin_hbm.at[idx_v.at[c]], sin_v.at[pl.ds(j * chunk, chunk)], sem))
            for cp in copies:
                cp.wait()
            pltpu.sync_copy(cos_v, cos_out.at[pl.ds(base + p * rows_pp, rows_pp)])
            pltpu.sync_copy(sin_v, sin_out.at[pl.ds(base + p * rows_pp, rows_pp)])

    return body(idx, cos_i, sin_i)


def _pack_table(tab):
    v, d = tab.shape
    return lax.bitcast_convert_type(tab.reshape(v, d // 2, 2), jnp.int32)


def _widen_body(cos_ref, sin_ref, cos_out, sin_out):
    bm, d = cos_out.shape

    def widen(x):
        a = jnp.repeat(x[:, : d // 2], 2, axis=1)
        b = jnp.repeat(x[:, d // 2:], 2, axis=1)
        odd = (lax.broadcasted_iota(jnp.int32, (bm // 2, d), 1) & 1) == 1
        lo = jnp.where(odd, (a >> 16) & 0xFFFF, a & 0xFFFF)
        hi = jnp.where(odd, jnp.bitwise_and(b, jnp.int32(-65536)), b << 16)
        return lo | hi

    cos_out.bitcast(jnp.int32)[...] = widen(cos_ref[...])
    sin_out.bitcast(jnp.int32)[...] = widen(sin_ref[...])


def _widen_to_f16(cos_r, sin_r, b, s, d, block_rows=2048):
    n2, _ = cos_r.shape
    n = 2 * n2
    grid = n // block_rows
    out_t = jax.ShapeDtypeStruct((n, d), jnp.float16)
    in_spec = pl.BlockSpec((block_rows // 2, d), lambda i: (i, 0))
    out_spec = pl.BlockSpec((block_rows, d), lambda i: (i, 0))
    return pl.pallas_call(
        _widen_body,
        grid=(grid,),
        in_specs=[in_spec, in_spec],
        out_specs=[out_spec, out_spec],
        out_shape=(out_t, out_t),
    )(cos_r, sin_r)


def kernel(position_ids, cos_cached, sin_cached):
    b, s = position_ids.shape
    v, d = cos_cached.shape
    n = b * s
    per_w = n // _NW
    n_ch = per_w // _CHUNK
    idx = position_ids.reshape(_NW, n_ch, _CHUNK)
    cos_i = _pack_table(cos_cached)
    sin_i = _pack_table(sin_cached)
    cos_r, sin_r = _gather_rows(idx, cos_i, sin_i, n_ch, _CHUNK, d // 2)
    cos_f, sin_f = _widen_to_f16(cos_r.reshape(n // 2, d),
                                 sin_r.reshape(n // 2, d), b, s, d)
    return cos_f.reshape(b, s, d), sin_f.reshape(b, s, d)


# speedup vs baseline: 10.4885x; 10.4885x over previous
import functools

import jax
import jax.numpy as jnp
from jax import lax
from jax.experimental import pallas as pl
from jax.experimental.pallas import tpu as pltpu
from jax.experimental.pallas import tpu_sc as plsc

_NUM_CORES = 2
_NUM_SUBCORES = 16
_NW = _NUM_CORES * _NUM_SUBCORES
_CHUNK = 128


@functools.partial(jax.jit, static_argnums=(3, 4, 5))
def _gather_rows(idx, cos_i, sin_i, n_ch, chunk, dw):
    n = _NW * n_ch * chunk
    per_w = n_ch * chunk
    n_pass = 2
    ch_pp = n_ch // n_pass
    rows_pp = ch_pp * chunk
    out_t = jax.ShapeDtypeStruct((n, dw), jnp.int32)
    mesh = plsc.VectorSubcoreMesh(core_axis_name="c", subcore_axis_name="s")

    @functools.partial(
        pl.kernel,
        mesh=mesh,
        out_type=(out_t, out_t),
        scratch_types=[
            pltpu.VMEM((n_ch, chunk), jnp.int32),
            pltpu.VMEM((rows_pp, dw), jnp.int32),
            pltpu.VMEM((rows_pp, dw), jnp.int32),
            pltpu.SemaphoreType.DMA,
        ],
        compiler_params=pltpu.CompilerParams(use_tc_tiling_on_sc=False),
    )
    def body(idx_hbm, cos_hbm, sin_hbm, cos_out, sin_out, idx_v, cos_v, sin_v, sem):
        wid = lax.axis_index("s") * _NUM_CORES + lax.axis_index("c")
        pltpu.sync_copy(idx_hbm.at[wid], idx_v)
        base = wid * per_w
        for p in range(n_pass):
            copies = []
            for j in range(ch_pp):
                c = p * ch_pp + j
                copies.append(pltpu.async_copy(
                    cos_hbm.at[idx_v.at[c]], cos_v.at[pl.ds(j * chunk, chunk)], sem))
                copies.append(pltpu.async_copy(
                    sin_hbm.at[idx_v.at[c]], sin_v.at[pl.ds(j * chunk, chunk)], sem))
            for cp in copies:
                cp.wait()
            pltpu.sync_copy(cos_v, cos_out.at[pl.ds(base + p * rows_pp, rows_pp)])
            pltpu.sync_copy(sin_v, sin_out.at[pl.ds(base + p * rows_pp, rows_pp)])

    return body(idx, cos_i, sin_i)


def _pack_table(tab):
    v, d = tab.shape
    return lax.bitcast_convert_type(
        jnp.repeat(tab, 2, axis=1).reshape(v, d, 2), jnp.int32)


def _widen_body(cos_ref, sin_ref, cos_out, sin_out):
    bm, d = cos_out.shape

    def widen(x):
        a = x[:, :d]
        b = x[:, d:]
        return (a & 0xFFFF) | jnp.bitwise_and(b, jnp.int32(-65536))

    cos_out.bitcast(jnp.int32)[...] = widen(cos_ref[...])
    sin_out.bitcast(jnp.int32)[...] = widen(sin_ref[...])


def _widen_to_f16(cos_r, sin_r, b, s, d, block_rows=2048):
    n2, _ = cos_r.shape
    n = 2 * n2
    grid = n // block_rows
    out_t = jax.ShapeDtypeStruct((n, d), jnp.float16)
    in_spec = pl.BlockSpec((block_rows // 2, 2 * d), lambda i: (i, 0))
    out_spec = pl.BlockSpec((block_rows, d), lambda i: (i, 0))
    return pl.pallas_call(
        _widen_body,
        grid=(grid,),
        in_specs=[in_spec, in_spec],
        out_specs=[out_spec, out_spec],
        out_shape=(out_t, out_t),
    )(cos_r, sin_r)


def kernel(position_ids, cos_cached, sin_cached):
    b, s = position_ids.shape
    v, d = cos_cached.shape
    n = b * s
    per_w = n // _NW
    n_ch = per_w // _CHUNK
    idx = position_ids.reshape(_NW, n_ch, _CHUNK)
    cos_i = _pack_table(cos_cached)
    sin_i = _pack_table(sin_cached)
    cos_r, sin_r = _gather_rows(idx, cos_i, sin_i, n_ch, _CHUNK, d)
    cos_f, sin_f = _widen_to_f16(cos_r.reshape(n // 2, 2 * d),
                                 sin_r.reshape(n // 2, 2 * d), b, s, d)
    return cos_f.reshape(b, s, d), sin_f.reshape(b, s, d)
